# baseline (speedup 1.0000x reference)
import functools

import jax
import jax.numpy as jnp
from jax import lax
from jax.experimental import pallas as pl
from jax.experimental.pallas import tpu as pltpu
from jax.experimental.pallas import tpu_sc as plsc

N_TOKENS = 16384
D_MODEL = 2048
N_EXPERTS = 16
K = 2
BLK = 2048
N_WORKERS = 32
RPW = N_TOKENS // N_WORKERS
GROUPS = RPW // 16


def _logits_block(x_ref, wt_ref, b_ref, eps_ref, nl_ref):
    logits = jnp.dot(x_ref[:], wt_ref[:], preferred_element_type=jnp.float32)
    logits = logits + b_ref[:]
    gate = logits[:, :N_EXPERTS]
    noisy_pre = logits[:, N_EXPERTS:]
    nl_ref[:] = gate + eps_ref[:] * jax.nn.softplus(noisy_pre)


def _tc_logits(x, wt, b, eps):
    grid = (N_TOKENS // BLK,)
    return pl.pallas_call(
        _logits_block,
        grid=grid,
        in_specs=[
            pl.BlockSpec((BLK, D_MODEL), lambda i: (i, 0)),
            pl.BlockSpec((D_MODEL, 2 * N_EXPERTS), lambda i: (0, 0)),
            pl.BlockSpec((1, 2 * N_EXPERTS), lambda i: (0, 0)),
            pl.BlockSpec((BLK, N_EXPERTS), lambda i: (i, 0)),
        ],
        out_specs=pl.BlockSpec((BLK, N_EXPERTS), lambda i: (i, 0)),
        out_shape=jax.ShapeDtypeStruct((N_TOKENS, N_EXPERTS), jnp.float32),
    )(x, wt, b, eps)


def _sc_route_body(nl_hbm, out_hbm, idx_hbm, nl_v, out_v, idx_v):
    c = lax.axis_index("c")
    s = lax.axis_index("s")
    wid = s * 2 + c
    pltpu.sync_copy(nl_hbm.at[pl.ds(wid * RPW * N_EXPERTS, RPW * N_EXPERTS)],
                    nl_v)

    lanes = lax.broadcasted_iota(jnp.int32, (16,), 0)

    def group_body(g, _):
        i1_vec = jnp.zeros((16,), jnp.int32)
        i2_vec = jnp.zeros((16,), jnp.int32)
        for r in range(16):
            off = (g * 16 + r) * N_EXPERTS
            row = nl_v[pl.ds(off, 16)]
            m1 = jnp.broadcast_to(jnp.max(row, axis=0), (16,))
            i1 = jnp.broadcast_to(
                jnp.min(jnp.where(row == m1, lanes, N_EXPERTS), axis=0), (16,))
            masked = jnp.where(lanes == i1, -jnp.inf, row)
            m2 = jnp.broadcast_to(jnp.max(masked, axis=0), (16,))
            i2 = jnp.broadcast_to(
                jnp.min(jnp.where(masked == m2, lanes, N_EXPERTS), axis=0),
                (16,))
            t = jnp.exp(m2 - m1)
            p1 = 1.0 / (1.0 + t)
            p2 = t * p1
            out_v[pl.ds(off, 16)] = jnp.where(
                lanes == i1, p1, jnp.where(lanes == i2, p2, 0.0))
            i1_vec = jnp.where(lanes == r, i1, i1_vec)
            i2_vec = jnp.where(lanes == r, i2, i2_vec)
        pairpos = (g * 16 + lanes) * 2
        plsc.store_scatter(idx_v, [pairpos], i1_vec)
        plsc.store_scatter(idx_v, [pairpos + 1], i2_vec)
        return 0

    lax.fori_loop(0, GROUPS, group_body, 0)
    pltpu.sync_copy(out_v,
                    out_hbm.at[pl.ds(wid * RPW * N_EXPERTS, RPW * N_EXPERTS)])
    pltpu.sync_copy(idx_v, idx_hbm.at[pl.ds(wid * RPW * K, RPW * K)])


_sc_route = functools.partial(
    pl.kernel,
    mesh=plsc.VectorSubcoreMesh(core_axis_name="c", subcore_axis_name="s"),
    compiler_params=pltpu.CompilerParams(needs_layout_passes=False),
    out_type=[
        jax.ShapeDtypeStruct((N_TOKENS * N_EXPERTS,), jnp.float32),
        jax.ShapeDtypeStruct((N_TOKENS * K,), jnp.int32),
    ],
    scratch_types=[
        pltpu.VMEM((RPW * N_EXPERTS,), jnp.float32),
        pltpu.VMEM((RPW * N_EXPERTS,), jnp.float32),
        pltpu.VMEM((RPW * K,), jnp.int32),
    ],
)(_sc_route_body)


def kernel(x, Wg, bg, Wn, bn):
    wt = jnp.concatenate([Wg, Wn], axis=0).T
    b = jnp.concatenate([bg, bn], axis=0)[None, :]
    eps = jax.random.normal(jax.random.key(42), (N_TOKENS, N_EXPERTS),
                            dtype=jnp.float32)
    nl = _tc_logits(x, wt, b, eps)
    router_flat, idx_flat = _sc_route(nl.reshape(-1))
    return (router_flat.reshape(N_TOKENS, N_EXPERTS),
            idx_flat.reshape(N_TOKENS, K))


# speedup vs baseline: 1.2025x; 1.2025x over previous
import functools

import jax
import jax.numpy as jnp
from jax import lax
from jax.experimental import pallas as pl
from jax.experimental.pallas import tpu as pltpu
from jax.experimental.pallas import tpu_sc as plsc

N_TOKENS = 16384
D_MODEL = 2048
N_EXPERTS = 16
K = 2
BLK = 2048
N_WORKERS = 32
RPW = N_TOKENS // N_WORKERS
GROUPS = RPW // 16


def _logits_block(x_ref, wt_ref, b_ref, eps_ref, nlt_ref):
    logits = jnp.dot(x_ref[:], wt_ref[:], preferred_element_type=jnp.float32)
    logits = logits + b_ref[:]
    gate = logits[:, :N_EXPERTS]
    noisy_pre = logits[:, N_EXPERTS:]
    nl = gate + eps_ref[:] * jax.nn.softplus(noisy_pre)
    nlt_ref[:] = nl.T


def _tc_logits_t(x, wt, b, eps):
    grid = (N_TOKENS // BLK,)
    return pl.pallas_call(
        _logits_block,
        grid=grid,
        in_specs=[
            pl.BlockSpec((BLK, D_MODEL), lambda i: (i, 0)),
            pl.BlockSpec((D_MODEL, 2 * N_EXPERTS), lambda i: (0, 0)),
            pl.BlockSpec((1, 2 * N_EXPERTS), lambda i: (0, 0)),
            pl.BlockSpec((BLK, N_EXPERTS), lambda i: (i, 0)),
        ],
        out_specs=pl.BlockSpec((N_EXPERTS, BLK), lambda i: (0, i)),
        out_shape=jax.ShapeDtypeStruct((N_EXPERTS, N_TOKENS), jnp.float32),
    )(x, wt, b, eps)


def _sc_route_body(nlt_hbm, out_hbm, idx_hbm, nlt_v, out_v, idx_v):
    c = lax.axis_index("c")
    s = lax.axis_index("s")
    wid = s * 2 + c
    base = wid * RPW
    pltpu.sync_copy(nlt_hbm.at[:, pl.ds(base, RPW)], nlt_v)

    lanes = lax.broadcasted_iota(jnp.int32, (16,), 0)
    neg_inf = jnp.full((16,), -jnp.inf, jnp.float32)
    zero = jnp.zeros((16,), jnp.float32)

    def group_body(g, _):
        vals = [nlt_v[e, pl.ds(g * 16, 16)] for e in range(N_EXPERTS)]
        m1 = vals[0]
        for e in range(1, N_EXPERTS):
            m1 = jnp.maximum(m1, vals[e])
        i1 = jnp.zeros((16,), jnp.int32)
        for e in range(N_EXPERTS - 1, -1, -1):
            i1 = jnp.where(vals[e] == m1, e, i1)
        m2 = neg_inf
        masked = []
        for e in range(N_EXPERTS):
            mv = jnp.where(i1 == e, neg_inf, vals[e])
            masked.append(mv)
            m2 = jnp.maximum(m2, mv)
        i2 = jnp.zeros((16,), jnp.int32)
        for e in range(N_EXPERTS - 1, -1, -1):
            i2 = jnp.where(masked[e] == m2, e, i2)
        t = jnp.exp(m2 - m1)
        p1 = 1.0 / (1.0 + t)
        p2 = t * p1
        tokpos = g * 16 + lanes
        outpos = tokpos * N_EXPERTS
        for e in range(N_EXPERTS):
            out_e = jnp.where(i1 == e, p1, jnp.where(i2 == e, p2, zero))
            plsc.store_scatter(out_v, [outpos + e], out_e)
        plsc.store_scatter(idx_v, [tokpos * 2], i1)
        plsc.store_scatter(idx_v, [tokpos * 2 + 1], i2)
        return 0

    lax.fori_loop(0, GROUPS, group_body, 0)
    pltpu.sync_copy(out_v, out_hbm.at[pl.ds(base * N_EXPERTS, RPW * N_EXPERTS)])
    pltpu.sync_copy(idx_v, idx_hbm.at[pl.ds(base * K, RPW * K)])


_sc_route = functools.partial(
    pl.kernel,
    mesh=plsc.VectorSubcoreMesh(core_axis_name="c", subcore_axis_name="s"),
    compiler_params=pltpu.CompilerParams(needs_layout_passes=False),
    out_type=[
        jax.ShapeDtypeStruct((N_TOKENS * N_EXPERTS,), jnp.float32),
        jax.ShapeDtypeStruct((N_TOKENS * K,), jnp.int32),
    ],
    scratch_types=[
        pltpu.VMEM((N_EXPERTS, RPW), jnp.float32),
        pltpu.VMEM((RPW * N_EXPERTS,), jnp.float32),
        pltpu.VMEM((RPW * K,), jnp.int32),
    ],
)(_sc_route_body)


def kernel(x, Wg, bg, Wn, bn):
    wt = jnp.concatenate([Wg, Wn], axis=0).T
    b = jnp.concatenate([bg, bn], axis=0)[None, :]
    eps = jax.random.normal(jax.random.key(42), (N_TOKENS, N_EXPERTS),
                            dtype=jnp.float32)
    nlt = _tc_logits_t(x, wt, b, eps)
    router_flat, idx_flat = _sc_route(nlt)
    return (router_flat.reshape(N_TOKENS, N_EXPERTS),
            idx_flat.reshape(N_TOKENS, K))
